# baseline (speedup 1.0000x reference)
import functools

import jax
import jax.numpy as jnp
from jax import lax
from jax.experimental import pallas as pl
from jax.experimental.pallas import tpu as pltpu
from jax.experimental.pallas import tpu_sc as plsc

_QB = 2048
_TW = 128
_OW = 128
_NC = 2
_NS = 16


def _table_kernel(x1_ref, f_ref, tbl_ref):
    keys = x1_ref[0] + f_ref[0]
    n1 = keys.shape[0]
    tbl_ref[0] = jnp.concatenate(
        [keys, f_ref[0], jnp.zeros((n1, _TW - 6), jnp.float32)], axis=1)


def _top3_kernel(q_ref, x1_ref, f_ref, idx_ref):
    b = pl.program_id(0)
    q = q_ref[0]
    keys = x1_ref[0] + f_ref[0]
    n1 = keys.shape[0]

    qb = (-2.0 * q).astype(jnp.bfloat16)
    kb = keys.astype(jnp.bfloat16)
    d = jnp.dot(qb, kb.T, preferred_element_type=jnp.float32)
    d = d + jnp.sum(q * q, axis=1, keepdims=True)
    d = d + jnp.sum(keys * keys, axis=1)[None, :]

    iota = jax.lax.broadcasted_iota(jnp.int32, d.shape, 1)

    idxs = []
    for r in range(3):
        m = jnp.min(d, axis=1, keepdims=True)
        im = jnp.min(jnp.where(d == m, iota, n1), axis=1, keepdims=True)
        idxs.append(im)
        if r < 2:
            d = jnp.where(iota == im, jnp.inf, d)

    idx_ref[0] = jnp.concatenate(idxs, axis=1) + b * n1


def _make_gather(n_rows, b_per_w, n_chunks):
    sub = b_per_w // n_chunks
    mesh = plsc.VectorSubcoreMesh(core_axis_name="c", subcore_axis_name="s")

    @functools.partial(
        pl.kernel, mesh=mesh,
        out_type=jax.ShapeDtypeStruct((n_rows, _OW), jnp.float32),
        scratch_types=[
            pltpu.VMEM((sub,), jnp.int32),
            pltpu.VMEM((sub,), jnp.int32),
            pltpu.VMEM((sub, _TW), jnp.float32),
            pltpu.VMEM((sub, _TW), jnp.float32),
            pltpu.SemaphoreType.DMA,
            pltpu.SemaphoreType.DMA,
        ],
    )
    def gather_rows(table_hbm, idx_hbm, out_hbm,
                    idx_a, idx_b, rows_a, rows_b, sem_a, sem_b):
        wid = lax.axis_index("s") * _NC + lax.axis_index("c")
        bufs = [(idx_a, rows_a, sem_a), (idx_b, rows_b, sem_b)]
        pending = [None, None]
        for t in range(n_chunks):
            idx_v, rows_v, sem = bufs[t % 2]
            base = wid * b_per_w + t * sub
            if pending[t % 2] is not None:
                p_handle, p_base = pending[t % 2]
                p_handle.wait()
                pltpu.sync_copy(rows_v, out_hbm.at[pl.ds(p_base, sub)])
            pltpu.sync_copy(idx_hbm.at[pl.ds(base, sub)], idx_v)
            pending[t % 2] = (
                pltpu.async_copy(table_hbm.at[idx_v], rows_v, sem), base)
        for t in (n_chunks - 2, n_chunks - 1):
            idx_v, rows_v, sem = bufs[t % 2]
            p_handle, p_base = pending[t % 2]
            p_handle.wait()
            pltpu.sync_copy(rows_v, out_hbm.at[pl.ds(p_base, sub)])

    return gather_rows


def _combine_kernel(q_ref, g_ref, out_ref):
    q = q_ref[0]
    n2 = q.shape[1]
    gt = jnp.transpose(g_ref[0][:, 0:8])
    qx, qy, qz = q[0:1, :], q[1:2, :], q[2:3, :]

    inv = []
    fx, fy, fz = [], [], []
    for k in range(3):
        o = k * n2
        dx = gt[0:1, o:o + n2] - qx
        dy = gt[1:2, o:o + n2] - qy
        dz = gt[2:3, o:o + n2] - qz
        sq = dx * dx + dy * dy + dz * dz
        dist = jnp.maximum(jnp.sqrt(sq), 1e-10)
        inv.append(1.0 / dist)
        fx.append(gt[3:4, o:o + n2])
        fy.append(gt[4:5, o:o + n2])
        fz.append(gt[5:6, o:o + n2])

    s = inv[0] + inv[1] + inv[2]
    w = [x / s for x in inv]
    flow2x = w[0] * fx[0] + w[1] * fx[1] + w[2] * fx[2]
    flow2y = w[0] * fy[0] + w[1] * fy[1] + w[2] * fy[2]
    flow2z = w[0] * fz[0] + w[1] * fz[1] + w[2] * fz[2]

    out_ref[0] = jnp.concatenate(
        [qx - flow2x, qy - flow2y, qz - flow2z], axis=0)


def kernel(xyz1, xyz2, flow1):
    b, c, n1 = xyz1.shape
    n2 = xyz2.shape[2]
    xyz1_t = jnp.transpose(xyz1, (0, 2, 1))
    flow1_t = jnp.transpose(flow1, (0, 2, 1))
    xyz2_t = jnp.transpose(xyz2, (0, 2, 1))

    table = pl.pallas_call(
        _table_kernel,
        grid=(b,),
        in_specs=[
            pl.BlockSpec((1, n1, c), lambda i: (i, 0, 0)),
            pl.BlockSpec((1, n1, c), lambda i: (i, 0, 0)),
        ],
        out_specs=pl.BlockSpec((1, n1, _TW), lambda i: (i, 0, 0)),
        out_shape=jax.ShapeDtypeStruct((b, n1, _TW), jnp.float32),
    )(xyz1_t, flow1_t)

    idx = pl.pallas_call(
        _top3_kernel,
        grid=(b, n2 // _QB),
        in_specs=[
            pl.BlockSpec((1, _QB, c), lambda i, j: (i, j, 0)),
            pl.BlockSpec((1, n1, c), lambda i, j: (i, 0, 0)),
            pl.BlockSpec((1, n1, c), lambda i, j: (i, 0, 0)),
        ],
        out_specs=pl.BlockSpec((1, _QB, 3), lambda i, j: (i, j, 0)),
        out_shape=jax.ShapeDtypeStruct((b, n2, 3), jnp.int32),
    )(xyz2_t, xyz1_t, flow1_t)

    idx_nm = jnp.transpose(idx, (0, 2, 1))
    n_rows = b * n2 * 3
    b_per_w = n_rows // (_NC * _NS)
    gathered = _make_gather(n_rows, b_per_w, 4)(
        table.reshape(b * n1, _TW), idx_nm.reshape(n_rows))

    g3 = gathered.reshape(b, 3 * n2, _OW)
    out = pl.pallas_call(
        _combine_kernel,
        grid=(b,),
        in_specs=[
            pl.BlockSpec((1, c, n2), lambda i: (i, 0, 0)),
            pl.BlockSpec((1, 3 * n2, _OW), lambda i: (i, 0, 0)),
        ],
        out_specs=pl.BlockSpec((1, c, n2), lambda i: (i, 0, 0)),
        out_shape=jax.ShapeDtypeStruct((b, c, n2), jnp.float32),
    )(xyz2, g3)

    return out


# speedup vs baseline: 1.1018x; 1.1018x over previous
import functools

import jax
import jax.numpy as jnp
from jax import lax
from jax.experimental import pallas as pl
from jax.experimental.pallas import tpu as pltpu
from jax.experimental.pallas import tpu_sc as plsc

_QB = 1024
_TW = 128
_OW = 128
_NC = 2
_NS = 16


def _table_kernel(x1_ref, f_ref, tbl_ref):
    keys = x1_ref[0] + f_ref[0]
    n1 = keys.shape[0]
    tbl_ref[0] = jnp.concatenate(
        [keys, f_ref[0], jnp.zeros((n1, _TW - 6), jnp.float32)], axis=1)


def _top3_kernel(q_ref, x1_ref, f_ref, idx_ref):
    b = pl.program_id(0)
    q = q_ref[0]
    keys = x1_ref[0] + f_ref[0]
    n1 = keys.shape[0]

    qb = (-2.0 * q).astype(jnp.bfloat16)
    kb = keys.astype(jnp.bfloat16)
    d = jnp.dot(qb, kb.T, preferred_element_type=jnp.float32)
    d = d + jnp.sum(q * q, axis=1, keepdims=True)
    d = d + jnp.sum(keys * keys, axis=1)[None, :]

    iota = jax.lax.broadcasted_iota(jnp.int32, d.shape, 1)

    idxs = []
    for r in range(3):
        im = jnp.argmin(d, axis=1).astype(jnp.int32)[:, None]
        idxs.append(im)
        if r < 2:
            d = jnp.where(iota == im, jnp.inf, d)

    idx_ref[0] = jnp.concatenate(idxs, axis=1) + b * n1


def _make_gather(n_rows, b_per_w, n_chunks):
    sub = b_per_w // n_chunks
    mesh = plsc.VectorSubcoreMesh(core_axis_name="c", subcore_axis_name="s")

    @functools.partial(
        pl.kernel, mesh=mesh,
        out_type=jax.ShapeDtypeStruct((n_rows, _OW), jnp.float32),
        scratch_types=[
            pltpu.VMEM((sub,), jnp.int32),
            pltpu.VMEM((sub,), jnp.int32),
            pltpu.VMEM((sub, _TW), jnp.float32),
            pltpu.VMEM((sub, _TW), jnp.float32),
            pltpu.SemaphoreType.DMA,
            pltpu.SemaphoreType.DMA,
        ],
    )
    def gather_rows(table_hbm, idx_hbm, out_hbm,
                    idx_a, idx_b, rows_a, rows_b, sem_a, sem_b):
        wid = lax.axis_index("s") * _NC + lax.axis_index("c")
        bufs = [(idx_a, rows_a, sem_a), (idx_b, rows_b, sem_b)]
        pending = [None, None]
        for t in range(n_chunks):
            idx_v, rows_v, sem = bufs[t % 2]
            base = wid * b_per_w + t * sub
            if pending[t % 2] is not None:
                p_handle, p_base = pending[t % 2]
                p_handle.wait()
                pltpu.sync_copy(rows_v, out_hbm.at[pl.ds(p_base, sub)])
            pltpu.sync_copy(idx_hbm.at[pl.ds(base, sub)], idx_v)
            pending[t % 2] = (
                pltpu.async_copy(table_hbm.at[idx_v], rows_v, sem), base)
        for t in (n_chunks - 2, n_chunks - 1):
            idx_v, rows_v, sem = bufs[t % 2]
            p_handle, p_base = pending[t % 2]
            p_handle.wait()
            pltpu.sync_copy(rows_v, out_hbm.at[pl.ds(p_base, sub)])

    return gather_rows


def _combine_kernel(q_ref, g_ref, out_ref):
    q = q_ref[0]
    n2 = q.shape[1]
    gt = jnp.transpose(g_ref[0][:, 0:8])
    qx, qy, qz = q[0:1, :], q[1:2, :], q[2:3, :]

    inv = []
    fx, fy, fz = [], [], []
    for k in range(3):
        o = k * n2
        dx = gt[0:1, o:o + n2] - qx
        dy = gt[1:2, o:o + n2] - qy
        dz = gt[2:3, o:o + n2] - qz
        sq = dx * dx + dy * dy + dz * dz
        dist = jnp.maximum(jnp.sqrt(sq), 1e-10)
        inv.append(1.0 / dist)
        fx.append(gt[3:4, o:o + n2])
        fy.append(gt[4:5, o:o + n2])
        fz.append(gt[5:6, o:o + n2])

    s = inv[0] + inv[1] + inv[2]
    w = [x / s for x in inv]
    flow2x = w[0] * fx[0] + w[1] * fx[1] + w[2] * fx[2]
    flow2y = w[0] * fy[0] + w[1] * fy[1] + w[2] * fy[2]
    flow2z = w[0] * fz[0] + w[1] * fz[1] + w[2] * fz[2]

    out_ref[0] = jnp.concatenate(
        [qx - flow2x, qy - flow2y, qz - flow2z], axis=0)


def kernel(xyz1, xyz2, flow1):
    b, c, n1 = xyz1.shape
    n2 = xyz2.shape[2]
    xyz1_t = jnp.transpose(xyz1, (0, 2, 1))
    flow1_t = jnp.transpose(flow1, (0, 2, 1))
    xyz2_t = jnp.transpose(xyz2, (0, 2, 1))

    table = pl.pallas_call(
        _table_kernel,
        grid=(b,),
        in_specs=[
            pl.BlockSpec((1, n1, c), lambda i: (i, 0, 0)),
            pl.BlockSpec((1, n1, c), lambda i: (i, 0, 0)),
        ],
        out_specs=pl.BlockSpec((1, n1, _TW), lambda i: (i, 0, 0)),
        out_shape=jax.ShapeDtypeStruct((b, n1, _TW), jnp.float32),
    )(xyz1_t, flow1_t)

    idx = pl.pallas_call(
        _top3_kernel,
        grid=(b, n2 // _QB),
        in_specs=[
            pl.BlockSpec((1, _QB, c), lambda i, j: (i, j, 0)),
            pl.BlockSpec((1, n1, c), lambda i, j: (i, 0, 0)),
            pl.BlockSpec((1, n1, c), lambda i, j: (i, 0, 0)),
        ],
        out_specs=pl.BlockSpec((1, _QB, 3), lambda i, j: (i, j, 0)),
        out_shape=jax.ShapeDtypeStruct((b, n2, 3), jnp.int32),
    )(xyz2_t, xyz1_t, flow1_t)

    idx_nm = jnp.transpose(idx, (0, 2, 1))
    n_rows = b * n2 * 3
    b_per_w = n_rows // (_NC * _NS)
    gathered = _make_gather(n_rows, b_per_w, 4)(
        table.reshape(b * n1, _TW), idx_nm.reshape(n_rows))

    g3 = gathered.reshape(b, 3 * n2, _OW)
    out = pl.pallas_call(
        _combine_kernel,
        grid=(b,),
        in_specs=[
            pl.BlockSpec((1, c, n2), lambda i: (i, 0, 0)),
            pl.BlockSpec((1, 3 * n2, _OW), lambda i: (i, 0, 0)),
        ],
        out_specs=pl.BlockSpec((1, c, n2), lambda i: (i, 0, 0)),
        out_shape=jax.ShapeDtypeStruct((b, c, n2), jnp.float32),
    )(xyz2, g3)

    return out
